# baseline (speedup 1.0000x reference)
import functools

import jax
import jax.numpy as jnp
from jax import lax
from jax.experimental import pallas as pl
from jax.experimental.pallas import tpu as pltpu
from jax.experimental.pallas import tpu_sc as plsc

_B = 16384
_D = 64
_K = 5
_NC = 2
_NS = 16
_NW = _NC * _NS
_PPW = _B // _NW
_CH = 64
_NCH = _PPW // _CH
_G = _CH // 16


def _log_sigmoid(x):
    e = jnp.exp(-jnp.abs(x))
    t = e / (e + 2.0)
    t2 = t * t
    p = jnp.float32(1.0 / 13.0)
    for c in (1.0 / 11.0, 1.0 / 9.0, 1.0 / 7.0, 1.0 / 5.0, 1.0 / 3.0, 1.0):
        p = p * t2 + jnp.float32(c)
    return jnp.minimum(x, 0.0) - 2.0 * t * p


def _body(idx_i, idx_j, idx_k, emb, out,
          idx_i_v, idx_j_v, idx_k_v, vi_v, vj_v, vk_v, obuf, sem):
    wid = lax.axis_index("s") * _NC + lax.axis_index("c")
    pair_base = wid * _PPW
    neg_base = pair_base * _K
    lanes = lax.iota(jnp.int32, 16)

    def chunk_body(c, carry):
        lp_acc, ln_acc = carry
        pb = pair_base + c * _CH
        nb = neg_base + c * (_CH * _K)
        pltpu.sync_copy(idx_i.at[pl.ds(pb, _CH)], idx_i_v)
        pltpu.sync_copy(idx_j.at[pl.ds(pb, _CH)], idx_j_v)
        pltpu.sync_copy(idx_k.at[pl.ds(nb, _CH * _K)], idx_k_v)
        pltpu.async_copy(emb.at[idx_i_v], vi_v, sem).wait()
        pltpu.async_copy(emb.at[idx_j_v], vj_v, sem).wait()
        pltpu.async_copy(emb.at[idx_k_v.at[pl.ds(0, 128)]],
                         vk_v.at[pl.ds(0, 128)], sem).wait()
        pltpu.async_copy(emb.at[idx_k_v.at[pl.ds(128, 128)]],
                         vk_v.at[pl.ds(128, 128)], sem).wait()
        pltpu.async_copy(emb.at[idx_k_v.at[pl.ds(256, 64)]],
                         vk_v.at[pl.ds(256, 64)], sem).wait()

        for g in range(_G):
            rows = g * 16 + lanes
            rows_k = [rows * _K + k for k in range(_K)]
            z = jnp.zeros((16,), jnp.float32)

            def dbody(d, accs, rows=rows, rows_k=rows_k):
                acc_p, acc_n = accs
                acc_n = list(acc_n)
                col = jnp.full((16,), d, jnp.int32)
                vi_c = plsc.load_gather(vi_v, [rows, col])
                vj_c = plsc.load_gather(vj_v, [rows, col])
                acc_p = acc_p + vi_c * vj_c
                for k in range(_K):
                    vk_c = plsc.load_gather(vk_v, [rows_k[k], col])
                    acc_n[k] = acc_n[k] + vi_c * vk_c
                return (acc_p, tuple(acc_n))

            acc_p, acc_n = lax.fori_loop(0, _D, dbody, (z, (z,) * _K),
                                         unroll=4)
            lp_acc = lp_acc + _log_sigmoid(acc_p)
            for k in range(_K):
                ln_acc = ln_acc + _log_sigmoid(-acc_n[k])
        return (lp_acc, ln_acc)

    z = jnp.zeros((16,), jnp.float32)
    lp_acc, ln_acc = lax.fori_loop(0, _NCH, chunk_body, (z, z))
    obuf[0, :] = lp_acc
    obuf[1, :] = ln_acc
    pltpu.sync_copy(obuf, out.at[wid])


@jax.jit
def kernel(node_pairs, neg_samples, embeddings):
    idx_i = node_pairs[:, 0].astype(jnp.int32)
    idx_j = node_pairs[:, 1].astype(jnp.int32)
    idx_k = neg_samples.reshape(-1).astype(jnp.int32)
    mesh = plsc.VectorSubcoreMesh(core_axis_name="c", subcore_axis_name="s")
    run = functools.partial(
        pl.kernel,
        mesh=mesh,
        compiler_params=pltpu.CompilerParams(
            needs_layout_passes=False, use_tc_tiling_on_sc=False),
        out_type=jax.ShapeDtypeStruct((_NW, 2, 16), jnp.float32),
        scratch_types=[
            pltpu.VMEM((_CH,), jnp.int32),
            pltpu.VMEM((_CH,), jnp.int32),
            pltpu.VMEM((_CH * _K,), jnp.int32),
            pltpu.VMEM((_CH, _D), jnp.float32),
            pltpu.VMEM((_CH, _D), jnp.float32),
            pltpu.VMEM((_CH * _K, _D), jnp.float32),
            pltpu.VMEM((2, 16), jnp.float32),
            pltpu.SemaphoreType.DMA,
        ],
    )(_body)
    parts = run(idx_i, idx_j, idx_k, embeddings)
    loss_pos = jnp.sum(parts[:, 0, :]) / _B
    loss_neg = jnp.sum(parts[:, 1, :]) / (_B * _K)
    return -(loss_pos + loss_neg)


# speedup vs baseline: 1.0301x; 1.0301x over previous
import functools

import jax
import jax.numpy as jnp
from jax import lax
from jax.experimental import pallas as pl
from jax.experimental.pallas import tpu as pltpu
from jax.experimental.pallas import tpu_sc as plsc

_B = 16384
_D = 64
_K = 5
_NC = 2
_NS = 16
_NW = _NC * _NS
_PPW = _B // _NW
_CH = 64
_NCH = _PPW // _CH
_G = _CH // 16
_NKC = _CH * _K


def _log_sigmoid(x):
    e = jnp.exp(-jnp.abs(x))
    t = e / (e + 2.0)
    t2 = t * t
    p = jnp.float32(1.0 / 13.0)
    for c in (1.0 / 11.0, 1.0 / 9.0, 1.0 / 7.0, 1.0 / 5.0, 1.0 / 3.0, 1.0):
        p = p * t2 + jnp.float32(c)
    return jnp.minimum(x, 0.0) - 2.0 * t * p


def _body(pairs, negs, emb, out,
          idx_p_v, idx_k_v, vij0, vij1, vk0, vk1, obuf, sem0, sem1):
    wid = lax.axis_index("s") * _NC + lax.axis_index("c")
    lanes = lax.iota(jnp.int32, 16)
    vij = (vij0, vij1)
    vk = (vk0, vk1)
    sem = (sem0, sem1)

    pltpu.sync_copy(pairs.at[pl.ds(wid * (2 * _PPW), 2 * _PPW)], idx_p_v)
    pltpu.sync_copy(negs.at[pl.ds(wid * (_PPW * _K), _PPW * _K)], idx_k_v)

    def gather_descs(c, par):
        kb = c * _NKC
        return (
            pltpu.make_async_copy(
                emb.at[idx_p_v.at[pl.ds(c * (2 * _CH), 2 * _CH)]],
                vij[par], sem[par]),
            pltpu.make_async_copy(
                emb.at[idx_k_v.at[pl.ds(kb, 128)]],
                vk[par].at[pl.ds(0, 128)], sem[par]),
            pltpu.make_async_copy(
                emb.at[idx_k_v.at[pl.ds(kb + 128, 128)]],
                vk[par].at[pl.ds(128, 128)], sem[par]),
            pltpu.make_async_copy(
                emb.at[idx_k_v.at[pl.ds(kb + 256, 64)]],
                vk[par].at[pl.ds(256, 64)], sem[par]),
        )

    def fire(c, par):
        for d in gather_descs(c, par):
            d.start()

    def drain(c, par):
        for d in gather_descs(c, par):
            d.wait()

    def compute(par, carry):
        lp_acc, ln_acc = carry
        vij_v = vij[par]
        vk_v = vk[par]
        for g in range(_G):
            rows = g * 16 + lanes
            rows_i = 2 * rows
            rows_j = rows_i + 1
            rows_k = [rows * _K + k for k in range(_K)]
            z = jnp.zeros((16,), jnp.float32)

            def dbody(d, accs, rows_i=rows_i, rows_j=rows_j, rows_k=rows_k):
                acc_p, acc_n = accs
                acc_n = list(acc_n)
                col = jnp.full((16,), d, jnp.int32)
                vi_c = plsc.load_gather(vij_v, [rows_i, col])
                vj_c = plsc.load_gather(vij_v, [rows_j, col])
                acc_p = acc_p + vi_c * vj_c
                for k in range(_K):
                    vk_c = plsc.load_gather(vk_v, [rows_k[k], col])
                    acc_n[k] = acc_n[k] + vi_c * vk_c
                return (acc_p, tuple(acc_n))

            acc_p, acc_n = lax.fori_loop(0, _D, dbody, (z, (z,) * _K),
                                         unroll=8)
            lp_acc = lp_acc + _log_sigmoid(acc_p)
            for k in range(_K):
                ln_acc = ln_acc + _log_sigmoid(-acc_n[k])
        return (lp_acc, ln_acc)

    fire(jnp.int32(0), 0)

    def pair_body(cc, carry):
        c0 = cc * 2
        c1 = c0 + 1
        fire(c1, 1)
        drain(c0, 0)
        carry = compute(0, carry)
        pl.when(c1 + 1 < _NCH)(lambda: fire(c1 + 1, 0))
        drain(c1, 1)
        carry = compute(1, carry)
        return carry

    z = jnp.zeros((16,), jnp.float32)
    lp_acc, ln_acc = lax.fori_loop(0, _NCH // 2, pair_body, (z, z))
    obuf[0, :] = lp_acc
    obuf[1, :] = ln_acc
    pltpu.sync_copy(obuf, out.at[wid])


@jax.jit
def kernel(node_pairs, neg_samples, embeddings):
    pairs_flat = node_pairs.reshape(-1).astype(jnp.int32)
    negs_flat = neg_samples.reshape(-1).astype(jnp.int32)
    mesh = plsc.VectorSubcoreMesh(core_axis_name="c", subcore_axis_name="s")
    run = functools.partial(
        pl.kernel,
        mesh=mesh,
        compiler_params=pltpu.CompilerParams(
            needs_layout_passes=False, use_tc_tiling_on_sc=False),
        out_type=jax.ShapeDtypeStruct((_NW, 2, 16), jnp.float32),
        scratch_types=[
            pltpu.VMEM((2 * _PPW,), jnp.int32),
            pltpu.VMEM((_PPW * _K,), jnp.int32),
            pltpu.VMEM((2 * _CH, _D), jnp.float32),
            pltpu.VMEM((2 * _CH, _D), jnp.float32),
            pltpu.VMEM((_NKC, _D), jnp.float32),
            pltpu.VMEM((_NKC, _D), jnp.float32),
            pltpu.VMEM((2, 16), jnp.float32),
            pltpu.SemaphoreType.DMA,
            pltpu.SemaphoreType.DMA,
        ],
    )(_body)
    parts = run(pairs_flat, negs_flat, embeddings)
    loss_pos = jnp.sum(parts[:, 0, :]) / _B
    loss_neg = jnp.sum(parts[:, 1, :]) / (_B * _K)
    return -(loss_pos + loss_neg)


# speedup vs baseline: 1.1837x; 1.1491x over previous
import functools

import jax
import jax.numpy as jnp
from jax import lax
from jax.experimental import pallas as pl
from jax.experimental.pallas import tpu as pltpu
from jax.experimental.pallas import tpu_sc as plsc

_B = 16384
_D = 64
_K = 5
_NC = 2
_NS = 16
_NW = _NC * _NS
_PPW = _B // _NW
_CH = 64
_NCH = _PPW // _CH
_G = _CH // 16
_NKC = _CH * _K


def _log_sigmoid(x):
    e = jnp.exp(-jnp.abs(x))
    t = e / (e + 2.0)
    t2 = t * t
    p = jnp.float32(1.0 / 13.0)
    for c in (1.0 / 11.0, 1.0 / 9.0, 1.0 / 7.0, 1.0 / 5.0, 1.0 / 3.0, 1.0):
        p = p * t2 + jnp.float32(c)
    return jnp.minimum(x, 0.0) - 2.0 * t * p


def _body(pairs, negs, emb, out,
          idx_p_v, idx_k_v, vij0, vij1, vk0, vk1, obuf, sem0, sem1):
    wid = lax.axis_index("s") * _NC + lax.axis_index("c")
    lanes = lax.iota(jnp.int32, 16)
    vij = (vij0, vij1)
    vk = (vk0, vk1)
    sem = (sem0, sem1)

    pltpu.sync_copy(pairs.at[pl.ds(wid * (2 * _PPW), 2 * _PPW)], idx_p_v)
    pltpu.sync_copy(negs.at[pl.ds(wid * (_PPW * _K), _PPW * _K)], idx_k_v)

    def gather_descs(c, par):
        kb = c * _NKC
        return (
            pltpu.make_async_copy(
                emb.at[idx_p_v.at[pl.ds(c * (2 * _CH), 2 * _CH)]],
                vij[par], sem[par]),
            pltpu.make_async_copy(
                emb.at[idx_k_v.at[pl.ds(kb, 128)]],
                vk[par].at[pl.ds(0, 128)], sem[par]),
            pltpu.make_async_copy(
                emb.at[idx_k_v.at[pl.ds(kb + 128, 128)]],
                vk[par].at[pl.ds(128, 128)], sem[par]),
            pltpu.make_async_copy(
                emb.at[idx_k_v.at[pl.ds(kb + 256, 64)]],
                vk[par].at[pl.ds(256, 64)], sem[par]),
        )

    def fire(c, par):
        for desc in gather_descs(c, par):
            desc.start()

    def drain(c, par):
        for desc in gather_descs(c, par):
            desc.wait()

    def compute(par, carry):
        lp_acc, ln_acc = carry
        vij_v = vij[par]
        vk_v = vk[par]
        for g in range(_G):
            rows = g * 16 + lanes
            rows_i = 2 * rows
            rows_j = rows_i + 1
            rows_k = [rows * _K + k for k in range(_K)]
            z = jnp.zeros((16,), jnp.float32)

            def dbody(d, accs, rows_i=rows_i, rows_j=rows_j, rows_k=rows_k):
                acc_p, acc_n = accs
                acc_n = list(acc_n)
                dvec = (jnp.full((16,), d, jnp.int32) + lanes) & (_D - 1)
                vi_c = plsc.load_gather(vij_v, [rows_i, dvec])
                vj_c = plsc.load_gather(vij_v, [rows_j, dvec])
                acc_p = acc_p + vi_c * vj_c
                for k in range(_K):
                    vk_c = plsc.load_gather(vk_v, [rows_k[k], dvec])
                    acc_n[k] = acc_n[k] + vi_c * vk_c
                return (acc_p, tuple(acc_n))

            acc_p, acc_n = lax.fori_loop(0, _D, dbody, (z, (z,) * _K),
                                         unroll=8)
            lp_acc = lp_acc + _log_sigmoid(acc_p)
            for k in range(_K):
                ln_acc = ln_acc + _log_sigmoid(-acc_n[k])
        return (lp_acc, ln_acc)

    fire(jnp.int32(0), 0)

    def pair_body(cc, carry):
        c0 = cc * 2
        c1 = c0 + 1
        fire(c1, 1)
        drain(c0, 0)
        carry = compute(0, carry)
        pl.when(c1 + 1 < _NCH)(lambda: fire(c1 + 1, 0))
        drain(c1, 1)
        carry = compute(1, carry)
        return carry

    z = jnp.zeros((16,), jnp.float32)
    lp_acc, ln_acc = lax.fori_loop(0, _NCH // 2, pair_body, (z, z))
    obuf[0, :] = lp_acc
    obuf[1, :] = ln_acc
    pltpu.sync_copy(obuf, out.at[wid])


@jax.jit
def kernel(node_pairs, neg_samples, embeddings):
    pairs_flat = node_pairs.reshape(-1).astype(jnp.int32)
    negs_flat = neg_samples.reshape(-1).astype(jnp.int32)

    mesh = plsc.VectorSubcoreMesh(core_axis_name="c", subcore_axis_name="s")
    run = functools.partial(
        pl.kernel,
        mesh=mesh,
        compiler_params=pltpu.CompilerParams(
            needs_layout_passes=False, use_tc_tiling_on_sc=False,
            disable_bounds_checks=True),
        out_type=jax.ShapeDtypeStruct((_NW, 2, 16), jnp.float32),
        scratch_types=[
            pltpu.VMEM((2 * _PPW,), jnp.int32),
            pltpu.VMEM((_PPW * _K,), jnp.int32),
            pltpu.VMEM((2 * _CH, _D), jnp.float32),
            pltpu.VMEM((2 * _CH, _D), jnp.float32),
            pltpu.VMEM((_NKC, _D), jnp.float32),
            pltpu.VMEM((_NKC, _D), jnp.float32),
            pltpu.VMEM((2, 16), jnp.float32),
            pltpu.SemaphoreType.DMA,
            pltpu.SemaphoreType.DMA,
        ],
    )(_body)
    parts = run(pairs_flat, negs_flat, embeddings)
    loss_pos = jnp.sum(parts[:, 0, :]) / _B
    loss_neg = jnp.sum(parts[:, 1, :]) / (_B * _K)
    return -(loss_pos + loss_neg)


# speedup vs baseline: 1.3122x; 1.1086x over previous
import functools

import jax
import jax.numpy as jnp
from jax import lax
from jax.experimental import pallas as pl
from jax.experimental.pallas import tpu as pltpu
from jax.experimental.pallas import tpu_sc as plsc

_B = 16384
_D = 64
_K = 5
_NC = 2
_NS = 16
_NW = _NC * _NS
_PPW = _B // _NW
_CH = 64
_NCH = _PPW // _CH
_G = _CH // 16
_NKC = _CH * _K


def _log_sigmoid(x):
    e = jnp.exp(-jnp.abs(x))
    t = e / (e + 2.0)
    t2 = t * t
    p = jnp.float32(1.0 / 13.0)
    for c in (1.0 / 11.0, 1.0 / 9.0, 1.0 / 7.0, 1.0 / 5.0, 1.0 / 3.0, 1.0):
        p = p * t2 + jnp.float32(c)
    return jnp.minimum(x, 0.0) - 2.0 * t * p


def _body(pairs, negs, emb, out,
          idx_p_v, idx_k_v, vij0, vij1, vk0, vk1, obuf, sem0, sem1):
    wid = lax.axis_index("s") * _NC + lax.axis_index("c")
    lanes = lax.iota(jnp.int32, 16)
    vij = (vij0, vij1)
    vk = (vk0, vk1)
    sem = (sem0, sem1)

    pltpu.sync_copy(pairs.at[pl.ds(wid * (2 * _PPW), 2 * _PPW)], idx_p_v)
    pltpu.sync_copy(negs.at[pl.ds(wid * (_PPW * _K), _PPW * _K)], idx_k_v)

    def gather_descs(c, par):
        kb = c * _NKC
        return (
            pltpu.make_async_copy(
                emb.at[idx_p_v.at[pl.ds(c * (2 * _CH), 2 * _CH)]],
                vij[par], sem[par]),
            pltpu.make_async_copy(
                emb.at[idx_k_v.at[pl.ds(kb, 128)]],
                vk[par].at[pl.ds(0, 128)], sem[par]),
            pltpu.make_async_copy(
                emb.at[idx_k_v.at[pl.ds(kb + 128, 128)]],
                vk[par].at[pl.ds(128, 128)], sem[par]),
            pltpu.make_async_copy(
                emb.at[idx_k_v.at[pl.ds(kb + 256, 64)]],
                vk[par].at[pl.ds(256, 64)], sem[par]),
        )

    def fire(c, par):
        for desc in gather_descs(c, par):
            desc.start()

    def drain(c, par):
        for desc in gather_descs(c, par):
            desc.wait()

    def compute(par, carry):
        lp_acc, ln_acc = carry
        vij_v = vij[par]
        vk_v = vk[par]
        for g in range(_G):
            rows = g * 16 + lanes
            rows_i = 2 * rows
            rows_j = rows_i + 1
            rows_k = [rows * _K + k for k in range(_K)]
            z = jnp.zeros((16,), jnp.float32)

            def dbody(d, accs, rows_i=rows_i, rows_j=rows_j, rows_k=rows_k):
                acc_p, acc_n = accs
                acc_n = list(acc_n)
                dvec = (jnp.full((16,), d, jnp.int32) + lanes) & (_D - 1)
                vi_c = plsc.load_gather(vij_v, [rows_i, dvec])
                vj_c = plsc.load_gather(vij_v, [rows_j, dvec])
                acc_p = acc_p + vi_c * vj_c
                for k in range(_K):
                    vk_c = plsc.load_gather(vk_v, [rows_k[k], dvec])
                    acc_n[k] = acc_n[k] + vi_c * vk_c
                return (acc_p, tuple(acc_n))

            acc_p, acc_n = lax.fori_loop(0, _D, dbody, (z, (z,) * _K),
                                         unroll=8)
            lp_acc = lp_acc + _log_sigmoid(acc_p)
            for k in range(_K):
                ln_acc = ln_acc + _log_sigmoid(-acc_n[k])
        return (lp_acc, ln_acc)

    fire(jnp.int32(0), 0)

    def pair_body(cc, carry):
        c0 = cc * 2
        c1 = c0 + 1
        fire(c1, 1)
        drain(c0, 0)
        carry = compute(0, carry)
        pl.when(c1 + 1 < _NCH)(lambda: fire(c1 + 1, 0))
        drain(c1, 1)
        carry = compute(1, carry)
        return carry

    z = jnp.zeros((16,), jnp.float32)
    lp_acc, ln_acc = lax.fori_loop(0, _NCH // 2, pair_body, (z, z))
    obuf[0, :] = lp_acc
    obuf[1, :] = ln_acc
    pltpu.sync_copy(obuf, out.at[wid])


@jax.jit
def kernel(node_pairs, neg_samples, embeddings):
    pairs_flat = node_pairs.reshape(-1).astype(jnp.int32)
    negs_flat = neg_samples.reshape(-1).astype(jnp.int32)
    emb_pad = jnp.pad(embeddings, ((0, 0), (0, _D)))
    mesh = plsc.VectorSubcoreMesh(core_axis_name="c", subcore_axis_name="s")
    run = functools.partial(
        pl.kernel,
        mesh=mesh,
        compiler_params=pltpu.CompilerParams(
            needs_layout_passes=False, disable_bounds_checks=True),
        out_type=jax.ShapeDtypeStruct((_NW, 2, 16), jnp.float32),
        scratch_types=[
            pltpu.VMEM((2 * _PPW,), jnp.int32),
            pltpu.VMEM((_PPW * _K,), jnp.int32),
            pltpu.VMEM((2 * _CH, 2 * _D), jnp.float32),
            pltpu.VMEM((2 * _CH, 2 * _D), jnp.float32),
            pltpu.VMEM((_NKC, 2 * _D), jnp.float32),
            pltpu.VMEM((_NKC, 2 * _D), jnp.float32),
            pltpu.VMEM((2, 16), jnp.float32),
            pltpu.SemaphoreType.DMA,
            pltpu.SemaphoreType.DMA,
        ],
    )(_body)
    parts = run(pairs_flat, negs_flat, emb_pad)
    loss_pos = jnp.sum(parts[:, 0, :]) / _B
    loss_neg = jnp.sum(parts[:, 1, :]) / (_B * _K)
    return -(loss_pos + loss_neg)
